# baseline (speedup 1.0000x reference)
import functools

import jax
import jax.numpy as jnp
from jax import lax
from jax.experimental import pallas as pl
from jax.experimental.pallas import tpu as pltpu
from jax.experimental.pallas import tpu_sc as plsc

_E = 320000
_D = 128
_TYPES = 512
_DELTA = 5.0
_SIGMA = 10.0

_NW = 32
_LANES = 16

_RBLK = 100
_EBLK = _RBLK * _D
_RPAD = 104
_SBLK = _RPAD * _D
_GRID = _E // _EBLK
_APAD = _GRID * _SBLK


def _sc_affine(edge_type, edge_distance, gain_table, offset_table):
    mesh = plsc.VectorSubcoreMesh(core_axis_name="c", subcore_axis_name="s")

    @functools.partial(
        pl.kernel,
        out_type=jax.ShapeDtypeStruct((_APAD,), jnp.float32),
        mesh=mesh,
        compiler_params=pltpu.CompilerParams(needs_layout_passes=False),
        scratch_types=[
            pltpu.VMEM((_EBLK,), jnp.int32),
            pltpu.VMEM((_EBLK,), jnp.float32),
            pltpu.VMEM((_EBLK,), jnp.float32),
            pltpu.VMEM((_TYPES,), jnp.float32),
            pltpu.VMEM((_TYPES,), jnp.float32),
            pltpu.SemaphoreType.DMA,
            pltpu.SemaphoreType.DMA,
        ],
    )
    def k(idx_hbm, x_hbm, gain_hbm, off_hbm, out_hbm,
          idx_v, x_v, a_v, gain_v, off_v, s_i, s_x):
        wid = lax.axis_index("s") * 2 + lax.axis_index("c")
        pltpu.sync_copy(gain_hbm, gain_v)
        pltpu.sync_copy(off_hbm, off_v)

        @pl.when(wid < _GRID)
        def _():
            b = wid
            in_i = (idx_hbm.at[pl.ds(b * _EBLK, _EBLK)], idx_v, s_i)
            in_x = (x_hbm.at[pl.ds(b * _EBLK, _EBLK)], x_v, s_x)
            pltpu.async_copy(*in_i)
            pltpu.async_copy(*in_x)
            pltpu.make_async_copy(*in_i).wait()
            pltpu.make_async_copy(*in_x).wait()

            def body(i, carry):
                sl = pl.ds(i * _LANES, _LANES)
                idx16 = idx_v[sl]
                g = plsc.load_gather(gain_v, [idx16])
                o = plsc.load_gather(off_v, [idx16])
                a_v[sl] = g * x_v[sl] + o
                return carry

            lax.fori_loop(0, _EBLK // _LANES, body, 0, unroll=8)
            pltpu.sync_copy(a_v, out_hbm.at[pl.ds(b * _SBLK, _EBLK)])

    return k(edge_type, edge_distance, gain_table, offset_table)


def _tc_body(a_ref, out_ref):
    s = 1.2011224087864498 / _SIGMA
    a = a_ref[0, :_RBLK, :] * s
    mu = lax.broadcasted_iota(jnp.int32, (1, 1, _D), 2).astype(jnp.float32) * (
        _DELTA / (_D - 1) * s
    )
    z = a[:, :, None] - mu
    out_ref[...] = jnp.exp2(z * z).reshape(_EBLK, _D)


def kernel(edge_type, edge_distance, gain_table, offset_table):
    a = _sc_affine(
        edge_type,
        edge_distance.reshape(_E),
        gain_table.reshape(_TYPES),
        offset_table.reshape(_TYPES),
    )
    return pl.pallas_call(
        _tc_body,
        grid=(_GRID,),
        in_specs=[pl.BlockSpec((1, _RPAD, _D), lambda i: (i, 0, 0))],
        out_specs=pl.BlockSpec((_EBLK, _D), lambda i: (i, 0)),
        out_shape=jax.ShapeDtypeStruct((_E, _D), jnp.float32),
    )(a.reshape(_GRID, _RPAD, _D))


# speedup vs baseline: 1.0276x; 1.0276x over previous
import functools

import jax
import jax.numpy as jnp
from jax import lax
from jax.experimental import pallas as pl
from jax.experimental.pallas import tpu as pltpu
from jax.experimental.pallas import tpu_sc as plsc

_E = 320000
_D = 128
_TYPES = 512
_DELTA = 5.0
_SIGMA = 10.0

_NW = 32
_LANES = 16

_RBLK = 100
_EBLK = _RBLK * _D
_RPAD = 104
_SBLK = _RPAD * _D
_GRID = _E // _EBLK
_APAD = _GRID * _SBLK


def _sc_affine(edge_type, edge_distance, gain_table, offset_table):
    mesh = plsc.VectorSubcoreMesh(core_axis_name="c", subcore_axis_name="s")

    @functools.partial(
        pl.kernel,
        out_type=jax.ShapeDtypeStruct((_APAD,), jnp.float32),
        mesh=mesh,
        compiler_params=pltpu.CompilerParams(needs_layout_passes=False),
        scratch_types=[
            pltpu.VMEM((_EBLK,), jnp.int32),
            pltpu.VMEM((_EBLK,), jnp.float32),
            pltpu.VMEM((_EBLK,), jnp.float32),
            pltpu.VMEM((_TYPES,), jnp.float32),
            pltpu.VMEM((_TYPES,), jnp.float32),
        ],
    )
    def k(idx_hbm, x_hbm, gain_hbm, off_hbm, out_hbm,
          idx_v, x_v, a_v, gain_v, off_v):
        wid = lax.axis_index("s") * 2 + lax.axis_index("c")
        pltpu.sync_copy(gain_hbm, gain_v)
        pltpu.sync_copy(off_hbm, off_v)

        @pl.when(wid < _GRID)
        def _():
            b = wid
            pltpu.sync_copy(idx_hbm.at[pl.ds(b * _EBLK, _EBLK)], idx_v)
            pltpu.sync_copy(x_hbm.at[pl.ds(b * _EBLK, _EBLK)], x_v)

            def body(i, carry):
                sl = pl.ds(i * _LANES, _LANES)
                idx16 = idx_v[sl]
                g = plsc.load_gather(gain_v, [idx16])
                o = plsc.load_gather(off_v, [idx16])
                a_v[sl] = g * x_v[sl] + o
                return carry

            lax.fori_loop(0, _EBLK // _LANES, body, 0)
            pltpu.sync_copy(a_v, out_hbm.at[pl.ds(b * _SBLK, _EBLK)])

    return k(edge_type, edge_distance, gain_table, offset_table)


def _tc_body(a_ref, out_ref):
    s = 1.2011224087864498 / _SIGMA
    a = a_ref[0, :_RBLK, :] * s
    mu = lax.broadcasted_iota(jnp.int32, (1, 1, _D), 2).astype(jnp.float32) * (
        _DELTA / (_D - 1) * s
    )
    z = a[:, :, None] - mu
    out_ref[...] = jnp.exp2(z * z).reshape(_EBLK, _D)


def kernel(edge_type, edge_distance, gain_table, offset_table):
    a = _sc_affine(
        edge_type,
        edge_distance.reshape(_E),
        gain_table.reshape(_TYPES),
        offset_table.reshape(_TYPES),
    )
    return pl.pallas_call(
        _tc_body,
        grid=(_GRID,),
        in_specs=[pl.BlockSpec((1, _RPAD, _D), lambda i: (i, 0, 0))],
        out_specs=pl.BlockSpec((_EBLK, _D), lambda i: (i, 0)),
        out_shape=jax.ShapeDtypeStruct((_E, _D), jnp.float32),
    )(a.reshape(_GRID, _RPAD, _D))
